# baseline (speedup 1.0000x reference)
import functools

import jax
import jax.numpy as jnp
from jax import lax
from jax.experimental import pallas as pl
from jax.experimental.pallas import tpu as pltpu
from jax.experimental.pallas import tpu_sc as plsc

NC = 2
NS = 16
NW = NC * NS
L = 16

B = 4096
F = 26
D = 16

BPW = B // NW
IPW = BPW * F
CHUNK = 128
NCHUNK = IPW // CHUNK


def _fm_body(x_hbm, xt_hbm, emb_hbm, lin_hbm, bias_hbm, out_hbm,
             idx_v, idxt_v, rows_v, lin_v, out_v, bias_v, sem_e, sem_l):
    c = lax.axis_index("c")
    s = lax.axis_index("s")
    wid = s * NC + c

    pltpu.sync_copy(x_hbm.at[wid], idx_v)
    pltpu.sync_copy(xt_hbm.at[wid], idxt_v)
    pltpu.sync_copy(bias_hbm, bias_v)

    def fire(j, carry):
        pltpu.make_async_copy(
            emb_hbm.at[idx_v.at[j]],
            rows_v.at[pl.ds(j * CHUNK, CHUNK)],
            sem_e,
        ).start()
        pltpu.make_async_copy(
            lin_hbm.at[idxt_v.at[j]],
            lin_v.at[j],
            sem_l,
        ).start()
        return carry

    lax.fori_loop(0, NCHUNK, fire, 0)
    pltpu.make_async_copy(emb_hbm.at[pl.ds(0, IPW)], rows_v, sem_e).wait()

    def drain_lin(j, carry):
        pltpu.make_async_copy(lin_hbm.at[pl.ds(0, CHUNK)], lin_v.at[j],
                              sem_l).wait()
        return carry

    lax.fori_loop(0, F, drain_lin, 0)

    iota = lax.iota(jnp.int32, L)
    p8 = jnp.bitwise_xor(iota, 8)
    p4 = jnp.bitwise_xor(iota, 4)
    p2 = jnp.bitwise_xor(iota, 2)
    p1 = jnp.bitwise_xor(iota, 1)
    bias_vec = bias_v[...]
    zero = jnp.zeros((L,), jnp.float32)

    def group(g, carry):
        base_r = g * L

        def row(r, acc):
            p0 = (base_r + r) * F
            v = rows_v[p0, :]
            s_acc = v
            ss_acc = v * v
            for f in range(1, F):
                v = rows_v[p0 + f, :]
                s_acc = s_acc + v
                ss_acc = ss_acc + v * v
            w = s_acc * s_acc - ss_acc
            w = w + w[p8]
            w = w + w[p4]
            w = w + w[p2]
            w = w + w[p1]
            return jnp.where(iota == r, w, acc)

        ix_vec = lax.fori_loop(0, L, row, zero)

        lin_vec = lin_v[0, pl.ds(base_r, L)]
        for f in range(1, F):
            lin_vec = lin_vec + lin_v[f, pl.ds(base_r, L)]

        z = ix_vec + lin_vec + bias_vec
        out_v[pl.ds(base_r, L)] = 1.0 / (1.0 + jnp.exp(-z))
        return carry

    lax.fori_loop(0, BPW // L, group, 0)

    pltpu.sync_copy(out_v, out_hbm.at[pl.ds(wid * BPW, BPW)])


@functools.partial(
    pl.kernel,
    out_type=jax.ShapeDtypeStruct((B,), jnp.float32),
    mesh=plsc.VectorSubcoreMesh(core_axis_name="c", subcore_axis_name="s"),
    scratch_types=[
        pltpu.VMEM((NCHUNK, CHUNK), jnp.int32),
        pltpu.VMEM((F, BPW), jnp.int32),
        pltpu.VMEM((IPW, D), jnp.float32),
        pltpu.VMEM((F, BPW), jnp.float32),
        pltpu.VMEM((BPW,), jnp.float32),
        pltpu.VMEM((L,), jnp.float32),
        pltpu.SemaphoreType.DMA,
        pltpu.SemaphoreType.DMA,
    ],
    compiler_params=pltpu.CompilerParams(use_tc_tiling_on_sc=False),
)
def _fm_kernel(x_hbm, xt_hbm, emb_hbm, lin_hbm, bias_hbm, out_hbm,
               idx_v, idxt_v, rows_v, lin_v, out_v, bias_v, sem_e, sem_l):
    _fm_body(x_hbm, xt_hbm, emb_hbm, lin_hbm, bias_hbm, out_hbm,
             idx_v, idxt_v, rows_v, lin_v, out_v, bias_v, sem_e, sem_l)


def kernel(x, linear_w, emb_w, bias):
    xi = x.astype(jnp.int32)
    x3 = xi.reshape(NW, NCHUNK, CHUNK)
    xt3 = xi.reshape(NW, BPW, F).transpose(0, 2, 1)
    lin = linear_w.reshape(-1)
    bias_vec = jnp.broadcast_to(bias.astype(jnp.float32), (L,))
    out = _fm_kernel(x3, xt3, emb_w, lin, bias_vec)
    return out.reshape(B, 1)


# speedup vs baseline: 1.0007x; 1.0007x over previous
import functools

import jax
import jax.numpy as jnp
from jax import lax
from jax.experimental import pallas as pl
from jax.experimental.pallas import tpu as pltpu
from jax.experimental.pallas import tpu_sc as plsc

NC = 2
NS = 16
NW = NC * NS
L = 16

B = 4096
F = 26
D = 16

BPW = B // NW
IPW = BPW * F
CHUNK = 128
NCHUNK = IPW // CHUNK
LPAD = 32


def _fm_body(x_hbm, emb_hbm, lin_hbm, bias_hbm, out_hbm,
             idx_v, rows_v, lin_v, out_v, bias_v, sem_e, sem_l):
    c = lax.axis_index("c")
    s = lax.axis_index("s")
    wid = s * NC + c

    pltpu.sync_copy(x_hbm.at[wid], idx_v)
    pltpu.sync_copy(bias_hbm, bias_v)

    def fire(j, carry):
        pltpu.make_async_copy(
            emb_hbm.at[idx_v.at[j]],
            rows_v.at[pl.ds(j * CHUNK, CHUNK)],
            sem_e,
        ).start()
        pltpu.make_async_copy(
            lin_hbm.at[idx_v.at[j]],
            lin_v.at[pl.ds(j * CHUNK, CHUNK)],
            sem_l,
        ).start()
        return carry

    lax.fori_loop(0, NCHUNK, fire, 0)
    pltpu.make_async_copy(emb_hbm.at[pl.ds(0, IPW)], rows_v, sem_e).wait()
    pltpu.make_async_copy(lin_hbm.at[pl.ds(0, IPW)],
                          lin_v.at[pl.ds(0, IPW)], sem_l).wait()

    iota = lax.iota(jnp.int32, L)
    p8 = jnp.bitwise_xor(iota, 8)
    p4 = jnp.bitwise_xor(iota, 4)
    p2 = jnp.bitwise_xor(iota, 2)
    p1 = jnp.bitwise_xor(iota, 1)
    tail_mask = iota < (F - L)
    bias_vec = bias_v[...]
    zero = jnp.zeros((L,), jnp.float32)

    def group(g, carry):
        base_r = g * L

        def row(r, acc):
            p0 = (base_r + r) * F
            v = rows_v[p0, :]
            s_acc = v
            ss_acc = v * v
            for f in range(1, F):
                v = rows_v[p0 + f, :]
                s_acc = s_acc + v
                ss_acc = ss_acc + v * v
            w = s_acc * s_acc - ss_acc
            l1 = lin_v[pl.ds(p0, L)]
            l2 = lin_v[pl.ds(p0 + L, L)]
            w = w + l1 + jnp.where(tail_mask, l2, zero)
            w = w + w[p8]
            w = w + w[p4]
            w = w + w[p2]
            w = w + w[p1]
            return jnp.where(iota == r, w, acc)

        z = lax.fori_loop(0, L, row, zero) + bias_vec
        out_v[pl.ds(base_r, L)] = 1.0 / (1.0 + jnp.exp(-z))
        return carry

    lax.fori_loop(0, BPW // L, group, 0)

    pltpu.sync_copy(out_v, out_hbm.at[pl.ds(wid * BPW, BPW)])


@functools.partial(
    pl.kernel,
    out_type=jax.ShapeDtypeStruct((B,), jnp.float32),
    mesh=plsc.VectorSubcoreMesh(core_axis_name="c", subcore_axis_name="s"),
    scratch_types=[
        pltpu.VMEM((NCHUNK, CHUNK), jnp.int32),
        pltpu.VMEM((IPW, D), jnp.float32),
        pltpu.VMEM((IPW + LPAD,), jnp.float32),
        pltpu.VMEM((BPW,), jnp.float32),
        pltpu.VMEM((L,), jnp.float32),
        pltpu.SemaphoreType.DMA,
        pltpu.SemaphoreType.DMA,
    ],
    compiler_params=pltpu.CompilerParams(use_tc_tiling_on_sc=False),
)
def _fm_kernel(x_hbm, emb_hbm, lin_hbm, bias_hbm, out_hbm,
               idx_v, rows_v, lin_v, out_v, bias_v, sem_e, sem_l):
    _fm_body(x_hbm, emb_hbm, lin_hbm, bias_hbm, out_hbm,
             idx_v, rows_v, lin_v, out_v, bias_v, sem_e, sem_l)


def kernel(x, linear_w, emb_w, bias):
    x3 = x.astype(jnp.int32).reshape(NW, NCHUNK, CHUNK)
    lin = linear_w.reshape(-1)
    bias_vec = jnp.broadcast_to(bias.astype(jnp.float32), (L,))
    out = _fm_kernel(x3, emb_w, lin, bias_vec)
    return out.reshape(B, 1)
